# baseline (speedup 1.0000x reference)
import functools

import jax
import jax.numpy as jnp
from jax import lax
from jax.experimental import pallas as pl
from jax.experimental.pallas import tpu as pltpu
from jax.experimental.pallas import tpu_sc as plsc

_N_LAYERS = 26
_VOCAB = 100000
_DIM = 32
_BATCH = 16384

_NUM_CORES = 2
_NUM_SUBCORES = 16
_NW = _NUM_CORES * _NUM_SUBCORES
_RPG = 128
_TOTAL_ROWS = _BATCH * _N_LAYERS
_PER_W = _TOTAL_ROWS // _NW
_STEPS = _PER_W // _RPG
_IDX_ROWS = _TOTAL_ROWS // _RPG
_TROWS = _N_LAYERS * _VOCAB * _DIM // 128
_NSLOT = 2
_NGRP = _STEPS // _NSLOT


def _make_sc_gather():
    mesh = plsc.VectorSubcoreMesh(core_axis_name="c", subcore_axis_name="s")

    @functools.partial(
        pl.kernel,
        mesh=mesh,
        out_type=jax.ShapeDtypeStruct((_TOTAL_ROWS, _DIM), jnp.float32),
        scratch_types=[
            pltpu.VMEM((_STEPS, _RPG), jnp.int32),
            pltpu.VMEM((_STEPS, _RPG), jnp.int32),
            [pltpu.VMEM((_RPG, 128), jnp.float32) for _ in range(_NSLOT)],
            [pltpu.VMEM((_RPG, _DIM), jnp.float32) for _ in range(_NSLOT)],
            [pltpu.SemaphoreType.DMA for _ in range(_NSLOT)],
            [pltpu.SemaphoreType.DMA for _ in range(_NSLOT)],
        ],
    )
    def gather_kernel(gidx_hbm, soff_hbm, table_hbm, out_hbm,
                      gidx_v, soff_v, bigs, rows, gsems, wsems):
        wid = lax.axis_index("s") * _NUM_CORES + lax.axis_index("c")
        idx_row0 = wid * _STEPS
        out_row0 = wid * _PER_W
        pltpu.sync_copy(gidx_hbm.at[pl.ds(idx_row0, _STEPS)], gidx_v)
        pltpu.sync_copy(soff_hbm.at[pl.ds(idx_row0, _STEPS)], soff_v)

        def l1_start(j, s):
            pltpu.async_copy(table_hbm.at[gidx_v.at[j]], bigs[s], gsems[s])

        def l1_wait(s):
            pltpu.make_async_copy(
                table_hbm.at[pl.ds(0, _RPG)], bigs[s], gsems[s]
            ).wait()

        def write_start(j, s):
            pltpu.async_copy(
                rows[s], out_hbm.at[pl.ds(out_row0 + j * _RPG, _RPG)], wsems[s]
            )

        def write_wait(s):
            pltpu.make_async_copy(
                rows[s], out_hbm.at[pl.ds(0, _RPG)], wsems[s]
            ).wait()

        def extract(j, s):
            big = bigs[s]
            row = rows[s]

            def ebody(it, carry):
                base = it * 16
                ovec = soff_v[j, pl.ds(base, 16)]
                for k in range(16):
                    i = base + k
                    off = ovec[k]
                    row[i, pl.ds(0, 16)] = big[i, pl.ds(off, 16)]
                    row[i, pl.ds(16, 16)] = big[i, pl.ds(off + 16, 16)]
                return carry

            lax.fori_loop(0, _RPG // 16, ebody, 0)

        for s in range(_NSLOT):
            l1_start(s, s)

        def body(g, carry):
            for s in range(_NSLOT):
                j = g * _NSLOT + s
                l1_wait(s)

                @pl.when(g > 0)
                def _():
                    write_wait(s)

                extract(j, s)

                @pl.when(g < _NGRP - 1)
                def _():
                    l1_start(j + _NSLOT, s)

                write_start(j, s)
            return carry

        lax.fori_loop(0, _NGRP, body, 0)

        for s in range(_NSLOT):
            write_wait(s)

    return gather_kernel


_sc_gather = _make_sc_gather()


def kernel(x, tables):
    offs = (jnp.arange(_N_LAYERS, dtype=jnp.int32) * _VOCAB)[None, :]
    flat = (x + offs).reshape(-1)
    gidx = (flat >> 2).reshape(_IDX_ROWS, _RPG)
    soff = ((flat & 3) << 5).reshape(_IDX_ROWS, _RPG)
    table2d = jnp.concatenate([tables[l] for l in range(_N_LAYERS)], axis=0)
    t128 = table2d.reshape(_TROWS, 128)
    out = _sc_gather(gidx, soff, t128)
    return out.reshape(_BATCH, _N_LAYERS * _DIM)


# speedup vs baseline: 2.1394x; 2.1394x over previous
import functools

import jax
import jax.numpy as jnp
from jax import lax
from jax.experimental import pallas as pl
from jax.experimental.pallas import tpu as pltpu
from jax.experimental.pallas import tpu_sc as plsc

_N_LAYERS = 26
_VOCAB = 100000
_DIM = 32
_BATCH = 16384

_NUM_CORES = 2
_NUM_SUBCORES = 16
_NW = _NUM_CORES * _NUM_SUBCORES
_ROWS_PER_GATHER = 128
_TOTAL_ROWS = _BATCH * _N_LAYERS
_PER_W = _TOTAL_ROWS // _NW
_STEPS = _PER_W // _ROWS_PER_GATHER
_IDX_ROWS = _TOTAL_ROWS // _ROWS_PER_GATHER
_NBUF = 4
_GROUP = _NBUF
_NK = _STEPS // (2 * _GROUP)


def _make_sc_gather():
    mesh = plsc.VectorSubcoreMesh(core_axis_name="c", subcore_axis_name="s")

    @functools.partial(
        pl.kernel,
        mesh=mesh,
        out_type=jax.ShapeDtypeStruct((_TOTAL_ROWS, _DIM), jnp.float32),
        scratch_types=[
            pltpu.VMEM((_STEPS, _ROWS_PER_GATHER), jnp.int32),
            pltpu.VMEM((2 * _NBUF, _ROWS_PER_GATHER, _DIM), jnp.float32),
            pltpu.SemaphoreType.DMA((2 * _NBUF,)),
            pltpu.SemaphoreType.DMA((2 * _NBUF,)),
        ],
        compiler_params=pltpu.CompilerParams(use_tc_tiling_on_sc=False),
    )
    def gather_kernel(idx_hbm, table_hbm, out_hbm, idx_v, rows_v, gsem, wsem):
        wid = lax.axis_index("s") * _NUM_CORES + lax.axis_index("c")
        idx_row0 = wid * _STEPS
        out_row0 = wid * _PER_W
        pltpu.sync_copy(idx_hbm.at[pl.ds(idx_row0, _STEPS)], idx_v)

        def gather_start(j, b):
            pltpu.async_copy(table_hbm.at[idx_v.at[j]], rows_v.at[b], gsem.at[b])

        def gather_wait(b):
            pltpu.make_async_copy(
                table_hbm.at[pl.ds(0, _ROWS_PER_GATHER)], rows_v.at[b], gsem.at[b]
            ).wait()

        def write_start(j, b):
            pltpu.async_copy(
                rows_v.at[b],
                out_hbm.at[pl.ds(out_row0 + j * _ROWS_PER_GATHER, _ROWS_PER_GATHER)],
                wsem.at[b],
            )

        def write_wait(b):
            pltpu.make_async_copy(
                rows_v.at[b], out_hbm.at[pl.ds(0, _ROWS_PER_GATHER)], wsem.at[b]
            ).wait()

        for i in range(_NBUF):
            gather_start(i, i)

        def body(k, carry):
            odd_base = (2 * k + 1) * _GROUP

            @pl.when(k > 0)
            def _():
                for i in range(_NBUF):
                    write_wait(_NBUF + i)

            for i in range(_NBUF):
                gather_start(odd_base + i, _NBUF + i)

            for i in range(_NBUF):
                gather_wait(i)
                write_start(2 * k * _GROUP + i, i)

            @pl.when(k < _NK - 1)
            def _():
                for i in range(_NBUF):
                    write_wait(i)
                    gather_start((2 * k + 2) * _GROUP + i, i)

            for i in range(_NBUF):
                gather_wait(_NBUF + i)
                write_start(odd_base + i, _NBUF + i)
            return carry

        lax.fori_loop(0, _NK, body, 0)

        for i in range(2 * _NBUF):
            write_wait(i)

    return gather_kernel


_sc_gather = _make_sc_gather()


def kernel(x, tables):
    offs = (jnp.arange(_N_LAYERS, dtype=jnp.int32) * _VOCAB)[None, :]
    flat_idx = (x + offs).reshape(_IDX_ROWS, _ROWS_PER_GATHER)
    table2d = tables.reshape(_N_LAYERS * _VOCAB, _DIM)
    out = _sc_gather(flat_idx, table2d)
    return out.reshape(_BATCH, _N_LAYERS * _DIM)


# speedup vs baseline: 2.1428x; 1.0016x over previous
import functools

import jax
import jax.numpy as jnp
from jax import lax
from jax.experimental import pallas as pl
from jax.experimental.pallas import tpu as pltpu
from jax.experimental.pallas import tpu_sc as plsc

_N_LAYERS = 26
_VOCAB = 100000
_DIM = 32
_BATCH = 16384

_NUM_CORES = 2
_NUM_SUBCORES = 16
_NW = _NUM_CORES * _NUM_SUBCORES
_ROWS_PER_GATHER = 128
_TOTAL_ROWS = _BATCH * _N_LAYERS
_PER_W = _TOTAL_ROWS // _NW
_STEPS = _PER_W // _ROWS_PER_GATHER
_IDX_ROWS = _TOTAL_ROWS // _ROWS_PER_GATHER
_NBUF = 4
_NK = _N_LAYERS // 2


def _make_sc_gather():
    mesh = plsc.VectorSubcoreMesh(core_axis_name="c", subcore_axis_name="s")

    @functools.partial(
        pl.kernel,
        mesh=mesh,
        out_type=jax.ShapeDtypeStruct((_BATCH, _N_LAYERS * _DIM), jnp.float32),
        scratch_types=[
            pltpu.VMEM((_N_LAYERS, _NBUF, _ROWS_PER_GATHER), jnp.int32),
            pltpu.VMEM((2 * _NBUF, _ROWS_PER_GATHER, _DIM), jnp.float32),
            pltpu.SemaphoreType.DMA((2 * _NBUF,)),
            pltpu.SemaphoreType.DMA((2 * _NBUF,)),
        ],
        compiler_params=pltpu.CompilerParams(use_tc_tiling_on_sc=False),
    )
    def gather_kernel(idx_hbm, table_hbm, out_hbm, idx_v, rows_v, gsem, wsem):
        wid = lax.axis_index("s") * _NUM_CORES + lax.axis_index("c")
        b_blk0 = wid * _NBUF
        pltpu.sync_copy(idx_hbm.at[:, pl.ds(b_blk0, _NBUF)], idx_v)

        def gather_start(l, i, b):
            pltpu.async_copy(
                table_hbm.at[idx_v.at[l, i]], rows_v.at[b], gsem.at[b]
            )

        def gather_wait(b):
            pltpu.make_async_copy(
                table_hbm.at[pl.ds(0, _ROWS_PER_GATHER)], rows_v.at[b], gsem.at[b]
            ).wait()

        def write_start(l, i, b):
            pltpu.async_copy(
                rows_v.at[b],
                out_hbm.at[
                    pl.ds((b_blk0 + i) * _ROWS_PER_GATHER, _ROWS_PER_GATHER),
                    pl.ds(l * _DIM, _DIM),
                ],
                wsem.at[b],
            )

        def write_wait(b):
            pltpu.make_async_copy(
                rows_v.at[b],
                out_hbm.at[pl.ds(0, _ROWS_PER_GATHER), pl.ds(0, _DIM)],
                wsem.at[b],
            ).wait()

        for i in range(_NBUF):
            gather_start(0, i, i)

        def body(k, carry):
            @pl.when(k > 0)
            def _():
                for i in range(_NBUF):
                    write_wait(_NBUF + i)

            for i in range(_NBUF):
                gather_start(2 * k + 1, i, _NBUF + i)

            for i in range(_NBUF):
                gather_wait(i)
                write_start(2 * k, i, i)

            @pl.when(k < _NK - 1)
            def _():
                for i in range(_NBUF):
                    write_wait(i)
                    gather_start(2 * k + 2, i, i)

            for i in range(_NBUF):
                gather_wait(_NBUF + i)
                write_start(2 * k + 1, i, _NBUF + i)
            return carry

        lax.fori_loop(0, _NK, body, 0)

        for i in range(2 * _NBUF):
            write_wait(i)

    return gather_kernel


_sc_gather = _make_sc_gather()


def kernel(x, tables):
    offs = (jnp.arange(_N_LAYERS, dtype=jnp.int32) * _VOCAB)[None, :]
    flat_idx = (x + offs).T.reshape(
        _N_LAYERS, _BATCH // _ROWS_PER_GATHER, _ROWS_PER_GATHER
    )
    table2d = tables.reshape(_N_LAYERS * _VOCAB, _DIM)
    return _sc_gather(flat_idx, table2d)


# speedup vs baseline: 2.1438x; 1.0005x over previous
import functools

import jax
import jax.numpy as jnp
from jax import lax
from jax.experimental import pallas as pl
from jax.experimental.pallas import tpu as pltpu
from jax.experimental.pallas import tpu_sc as plsc

_N_LAYERS = 26
_VOCAB = 100000
_DIM = 32
_BATCH = 16384

_NUM_CORES = 2
_NUM_SUBCORES = 16
_NW = _NUM_CORES * _NUM_SUBCORES
_ROWS_PER_GATHER = 128
_TOTAL_ROWS = _BATCH * _N_LAYERS
_NBUF = 4
_NK = _N_LAYERS // 2


def _make_sc_gather():
    mesh = plsc.VectorSubcoreMesh(core_axis_name="c", subcore_axis_name="s")

    @functools.partial(
        pl.kernel,
        mesh=mesh,
        out_type=jax.ShapeDtypeStruct((_BATCH, _N_LAYERS * _DIM), jnp.float32),
        scratch_types=[
            pltpu.VMEM((_N_LAYERS, _NBUF, _ROWS_PER_GATHER), jnp.int32),
            pltpu.VMEM((2 * _NBUF, _ROWS_PER_GATHER, _DIM), jnp.float32),
            pltpu.SemaphoreType.DMA((2 * _NBUF,)),
            pltpu.SemaphoreType.DMA((2 * _NBUF,)),
        ],
        compiler_params=pltpu.CompilerParams(use_tc_tiling_on_sc=False),
    )
    def gather_kernel(idx_hbm, table_hbm, out_hbm, idx_v, rows_v, gsem, wsem):
        wid = lax.axis_index("s") * _NUM_CORES + lax.axis_index("c")
        b_blk0 = wid * _NBUF
        pltpu.sync_copy(idx_hbm.at[:, pl.ds(b_blk0, _NBUF)], idx_v)

        def gather_start(l, i, b):
            pltpu.async_copy(
                table_hbm.at[idx_v.at[l, i]], rows_v.at[b], gsem.at[b]
            )

        def gather_wait(b):
            pltpu.make_async_copy(
                table_hbm.at[pl.ds(0, _ROWS_PER_GATHER)], rows_v.at[b], gsem.at[b]
            ).wait()

        def write_start(l, i, b):
            pltpu.async_copy(
                rows_v.at[b],
                out_hbm.at[
                    pl.ds((b_blk0 + i) * _ROWS_PER_GATHER, _ROWS_PER_GATHER),
                    pl.ds(l * _DIM, _DIM),
                ],
                wsem.at[b],
            )

        def write_wait(b):
            pltpu.make_async_copy(
                rows_v.at[b],
                out_hbm.at[pl.ds(0, _ROWS_PER_GATHER), pl.ds(0, _DIM)],
                wsem.at[b],
            ).wait()

        for i in range(_NBUF):
            gather_start(0, i, i)

        def body(k, carry):
            @pl.when(k > 0)
            def _():
                for i in range(_NBUF):
                    write_wait(_NBUF + i)

            for i in range(_NBUF):
                gather_start(2 * k + 1, i, _NBUF + i)

            for i in range(_NBUF):
                gather_wait(i)
                write_start(2 * k, i, i)

            @pl.when(k < _NK - 1)
            def _():
                for i in range(_NBUF):
                    write_wait(i)
                    gather_start(2 * k + 2, i, i)

            for i in range(_NBUF):
                gather_wait(_NBUF + i)
                write_start(2 * k + 1, i, _NBUF + i)
            return carry

        lax.fori_loop(0, _NK, body, 0)

        for i in range(2 * _NBUF):
            write_wait(i)

    return gather_kernel


_sc_gather = _make_sc_gather()


def kernel(x, tables):
    offs = (jnp.arange(_N_LAYERS, dtype=jnp.int32) * _VOCAB)[None, :]
    flat_idx = (x + offs).T.reshape(
        _N_LAYERS, _BATCH // _ROWS_PER_GATHER, _ROWS_PER_GATHER
    )
    table2d = tables.reshape(_N_LAYERS * _VOCAB, _DIM)
    return _sc_gather(flat_idx, table2d)
